# baseline (speedup 1.0000x reference)
import jax
import jax.numpy as jnp
from jax import lax
from jax.experimental import pallas as pl
from jax.experimental.pallas import tpu as pltpu
from jax.experimental.pallas import tpu_sc as plsc

_B = 16384
_EMB = 32
_N = 1000000
_NW = 32
_BPW = _B // _NW
_CHUNK = 128
_NQ = _BPW * _EMB // _CHUNK
_WAVES = 4


def _sc_body(users_hbm, items_hbm, uflat_hbm, iflat_hbm, bias_hbm, out_hbm,
             uidx_v, iidx_v, bidx, uw_idx, iw_idx, ubuf, ibuf, bias_v, out_v,
             sem, bsem):
    wid = lax.axis_index("s") * 2 + lax.axis_index("c")
    base = wid * _BPW

    pltpu.sync_copy(users_hbm.at[pl.ds(base, _BPW)], uidx_v)
    pltpu.sync_copy(items_hbm.at[pl.ds(base, _BPW)], iidx_v)

    for c in range(_BPW // 16):
        v = iidx_v[pl.ds(c * 16, 16)]
        cv = jnp.where(v == jnp.int32(-1), jnp.int32(0), v)
        bidx[c * 16 // _CHUNK, pl.ds((c * 16) % _CHUNK, 16)] = cv

    bias_copies = [pltpu.make_async_copy(
        bias_hbm.at[bidx.at[j]], bias_v.at[pl.ds(j * _CHUNK, _CHUNK)], bsem)
        for j in range(_BPW // _CHUNK)]
    for cp in bias_copies:
        cp.start()

    def build(m, carry):
        rvec_u = uidx_v[pl.ds(m * 16, 16)]
        ri = iidx_v[pl.ds(m * 16, 16)]
        rvec_i = jnp.where(ri == jnp.int32(-1), jnp.int32(0), ri)
        for o in range(4):
            for k in range(8):
                d = o * 8 + k
                uw_idx[m * 4 + o, pl.ds(k * 16, 16)] = rvec_u + jnp.int32(
                    d * _N)
                iw_idx[m * 4 + o, pl.ds(k * 16, 16)] = rvec_i + jnp.int32(
                    d * _N)
        return carry

    lax.fori_loop(0, _BPW // 16, build, 0)

    wave_chunks = _NQ // _WAVES
    wave_words = wave_chunks * _CHUNK
    for w in range(_WAVES):
        for qq in range(wave_chunks):
            q = w * wave_chunks + qq
            pltpu.async_copy(uflat_hbm.at[uw_idx.at[q]],
                             ubuf.at[pl.ds(q * _CHUNK, _CHUNK)], sem)
            pltpu.async_copy(iflat_hbm.at[iw_idx.at[q]],
                             ibuf.at[pl.ds(q * _CHUNK, _CHUNK)], sem)
        pltpu.make_async_copy(
            out_hbm.at[pl.ds(0, wave_words)],
            ubuf.at[pl.ds(w * wave_words, wave_words)], sem).wait()
        pltpu.make_async_copy(
            out_hbm.at[pl.ds(0, wave_words)],
            ibuf.at[pl.ds(w * wave_words, wave_words)], sem).wait()

    for cp in bias_copies:
        cp.wait()

    def chunk(m, carry):
        sl = pl.ds(m * 16, 16)
        acc = bias_v[sl]
        for o in range(4):
            for k in range(8):
                psl = pl.ds((m * 4 + o) * _CHUNK + k * 16, 16)
                acc = acc + ubuf[psl] * ibuf[psl]
        it = iidx_v[sl]
        out_v[sl] = jnp.where(it == jnp.int32(-1), jnp.float32(-100.0), acc)
        return carry

    lax.fori_loop(0, _BPW // 16, chunk, 0)

    pltpu.sync_copy(out_v, out_hbm.at[pl.ds(base, _BPW)])


@jax.jit
def _sc_call(users, items, uflat, iflat, bias_flat):
    mesh = plsc.VectorSubcoreMesh(core_axis_name="c", subcore_axis_name="s")
    params = pltpu.CompilerParams(
        needs_layout_passes=False, use_tc_tiling_on_sc=False)
    out = pl.kernel(
        _sc_body,
        mesh=mesh,
        compiler_params=params,
        out_type=jax.ShapeDtypeStruct((_B,), jnp.float32),
        scratch_types=[
            pltpu.VMEM((_BPW,), jnp.int32),
            pltpu.VMEM((_BPW,), jnp.int32),
            pltpu.VMEM((_BPW // _CHUNK, _CHUNK), jnp.int32),
            pltpu.VMEM((_NQ, _CHUNK), jnp.int32),
            pltpu.VMEM((_NQ, _CHUNK), jnp.int32),
            pltpu.VMEM((_BPW * _EMB,), jnp.float32),
            pltpu.VMEM((_BPW * _EMB,), jnp.float32),
            pltpu.VMEM((_BPW,), jnp.float32),
            pltpu.VMEM((_BPW,), jnp.float32),
            pltpu.SemaphoreType.DMA,
            pltpu.SemaphoreType.DMA,
        ],
    )(users, items, uflat, iflat, bias_flat)
    return out


def kernel(users, items, user_table, item_table, bias_table):
    users32 = users.astype(jnp.int32)
    items32 = items.astype(jnp.int32)
    uflat = jnp.reshape(user_table.T, (-1,))
    iflat = jnp.reshape(item_table.T, (-1,))
    bias_flat = jnp.reshape(bias_table, (-1,))
    return _sc_call(users32, items32, uflat, iflat, bias_flat)


# speedup vs baseline: 4.9162x; 4.9162x over previous
import jax
import jax.numpy as jnp
from jax import lax
from jax.experimental import pallas as pl
from jax.experimental.pallas import tpu as pltpu
from jax.experimental.pallas import tpu_sc as plsc

_B = 16384
_EMB = 32
_NW = 32
_BPW = _B // _NW
_CHUNK = 128
_NCH = _BPW // _CHUNK


def _sc_body(users_hbm, items_hbm, utab_hbm, itab_hbm, bias_hbm, out_hbm,
             uidx, iidx_raw, iidx_clean, urows_bf, irows_bf, urows, irows,
             bias_v, out_v, sem):
    wid = lax.axis_index("s") * 2 + lax.axis_index("c")
    base = wid * _BPW

    for j in range(_NCH):
        pltpu.sync_copy(users_hbm.at[pl.ds(base + j * _CHUNK, _CHUNK)],
                        uidx.at[j])
    pltpu.sync_copy(items_hbm.at[pl.ds(base, _BPW)], iidx_raw)

    for c in range(_BPW // 16):
        v = iidx_raw[pl.ds(c * 16, 16)]
        cv = jnp.where(v == jnp.int32(-1), jnp.int32(0), v)
        iidx_clean[c // (_CHUNK // 16), pl.ds((c % (_CHUNK // 16)) * 16, 16)] = cv

    copies = []
    for j in range(_NCH):
        sl = pl.ds(j * _CHUNK, _CHUNK)
        copies.append(pltpu.make_async_copy(
            utab_hbm.at[uidx.at[j]], urows_bf.at[sl], sem))
        copies.append(pltpu.make_async_copy(
            itab_hbm.at[iidx_clean.at[j]], irows_bf.at[sl], sem))
        copies.append(pltpu.make_async_copy(
            bias_hbm.at[iidx_clean.at[j]], bias_v.at[sl], sem))
    for cp in copies:
        cp.start()
    for cp in copies:
        cp.wait()

    def widen(r, carry):
        ur = urows_bf[r, :]
        ua, ub = plsc.unpack(ur, format=plsc.PackFormat.INTERLEAVED)
        urows[r, pl.ds(0, 16)] = ua
        urows[r, pl.ds(16, 16)] = ub
        ir = irows_bf[r, :]
        ia, ib = plsc.unpack(ir, format=plsc.PackFormat.INTERLEAVED)
        irows[r, pl.ds(0, 16)] = ia
        irows[r, pl.ds(16, 16)] = ib
        return carry

    lax.fori_loop(0, _BPW, widen, 0)

    lane = lax.iota(jnp.int32, 16)
    cols = [(lane + t) % _EMB for t in range(_EMB)]

    def chunk(c, carry):
        rows16 = c * 16 + lane
        acc = bias_v[pl.ds(c * 16, 16)]
        for t in range(_EMB):
            u = plsc.load_gather(urows, [rows16, cols[t]])
            v = plsc.load_gather(irows, [rows16, cols[t]])
            acc = acc + u * v
        it = iidx_raw[pl.ds(c * 16, 16)]
        res = jnp.where(it == jnp.int32(-1), jnp.float32(-100.0), acc)
        out_v[pl.ds(c * 16, 16)] = res
        return carry

    lax.fori_loop(0, _BPW // 16, chunk, 0)

    pltpu.sync_copy(out_v, out_hbm.at[pl.ds(base, _BPW)])


@jax.jit
def _sc_call(users, items, user_table, item_table, bias_flat):
    mesh = plsc.VectorSubcoreMesh(core_axis_name="c", subcore_axis_name="s")
    f = pl.kernel(
        _sc_body,
        mesh=mesh,
        compiler_params=pltpu.CompilerParams(
            needs_layout_passes=False, use_tc_tiling_on_sc=False),
        out_type=jax.ShapeDtypeStruct((_B,), jnp.float32),
        scratch_types=[
            pltpu.VMEM((_NCH, _CHUNK), jnp.int32),
            pltpu.VMEM((_BPW,), jnp.int32),
            pltpu.VMEM((_NCH, _CHUNK), jnp.int32),
            pltpu.VMEM((_BPW, _EMB), jnp.bfloat16),
            pltpu.VMEM((_BPW, _EMB), jnp.bfloat16),
            pltpu.VMEM((_BPW, _EMB), jnp.float32),
            pltpu.VMEM((_BPW, _EMB), jnp.float32),
            pltpu.VMEM((_BPW,), jnp.float32),
            pltpu.VMEM((_BPW,), jnp.float32),
            pltpu.SemaphoreType.DMA,
        ],
    )
    return f(users, items, user_table, item_table, bias_flat)


def kernel(users, items, user_table, item_table, bias_table):
    users32 = users.astype(jnp.int32)
    items32 = items.astype(jnp.int32)
    bias_flat = jnp.reshape(bias_table, (-1,))
    utab_bf = user_table.astype(jnp.bfloat16)
    itab_bf = item_table.astype(jnp.bfloat16)
    return _sc_call(users32, items32, utab_bf, itab_bf, bias_flat)


# speedup vs baseline: 5.7735x; 1.1744x over previous
import jax
import jax.numpy as jnp
from jax import lax
from jax.experimental import pallas as pl
from jax.experimental.pallas import tpu as pltpu
from jax.experimental.pallas import tpu_sc as plsc

_B = 16384
_EMB = 32
_NW = 32
_BPW = _B // _NW
_CHUNK = 128
_NCH = _BPW // _CHUNK


def _sc_body(users_hbm, items_hbm, utab_hbm, itab_hbm, bias_hbm, out_hbm,
             uidx, iidx_raw, iidx_clean, urows, irows, bias_v, out_v, sem):
    wid = lax.axis_index("s") * 2 + lax.axis_index("c")
    base = wid * _BPW

    for j in range(_NCH):
        pltpu.sync_copy(users_hbm.at[pl.ds(base + j * _CHUNK, _CHUNK)],
                        uidx.at[j])
    pltpu.sync_copy(items_hbm.at[pl.ds(base, _BPW)], iidx_raw)

    for c in range(_BPW // 16):
        v = iidx_raw[pl.ds(c * 16, 16)]
        cv = jnp.where(v == jnp.int32(-1), jnp.int32(0), v)
        iidx_clean[c // (_CHUNK // 16), pl.ds((c % (_CHUNK // 16)) * 16, 16)] = cv

    copies = []
    for j in range(_NCH):
        sl = pl.ds(j * _CHUNK, _CHUNK)
        copies.append(pltpu.make_async_copy(
            utab_hbm.at[uidx.at[j]], urows.at[sl], sem))
        copies.append(pltpu.make_async_copy(
            itab_hbm.at[iidx_clean.at[j]], irows.at[sl], sem))
        copies.append(pltpu.make_async_copy(
            bias_hbm.at[iidx_clean.at[j]], bias_v.at[sl], sem))
    for cp in copies:
        cp.start()
    for cp in copies:
        cp.wait()

    lane = lax.iota(jnp.int32, 16)
    cols = [(lane + t) % _EMB for t in range(_EMB)]

    def chunk(c, carry):
        rows16 = c * 16 + lane
        acc = bias_v[pl.ds(c * 16, 16)]
        for t in range(_EMB):
            u = plsc.load_gather(urows, [rows16, cols[t]])
            v = plsc.load_gather(irows, [rows16, cols[t]])
            acc = acc + u * v
        it = iidx_raw[pl.ds(c * 16, 16)]
        res = jnp.where(it == jnp.int32(-1), jnp.float32(-100.0), acc)
        out_v[pl.ds(c * 16, 16)] = res
        return carry

    lax.fori_loop(0, _BPW // 16, chunk, 0)

    pltpu.sync_copy(out_v, out_hbm.at[pl.ds(base, _BPW)])


@jax.jit
def _sc_call(users, items, user_table, item_table, bias_flat):
    mesh = plsc.VectorSubcoreMesh(core_axis_name="c", subcore_axis_name="s")
    f = pl.kernel(
        _sc_body,
        mesh=mesh,
        compiler_params=pltpu.CompilerParams(
            needs_layout_passes=False, use_tc_tiling_on_sc=False),
        out_type=jax.ShapeDtypeStruct((_B,), jnp.float32),
        scratch_types=[
            pltpu.VMEM((_NCH, _CHUNK), jnp.int32),
            pltpu.VMEM((_BPW,), jnp.int32),
            pltpu.VMEM((_NCH, _CHUNK), jnp.int32),
            pltpu.VMEM((_BPW, _EMB), jnp.float32),
            pltpu.VMEM((_BPW, _EMB), jnp.float32),
            pltpu.VMEM((_BPW,), jnp.float32),
            pltpu.VMEM((_BPW,), jnp.float32),
            pltpu.SemaphoreType.DMA,
        ],
    )
    return f(users, items, user_table, item_table, bias_flat)


def kernel(users, items, user_table, item_table, bias_table):
    users32 = users.astype(jnp.int32)
    items32 = items.astype(jnp.int32)
    bias_flat = jnp.reshape(bias_table, (-1,))
    return _sc_call(users32, items32, user_table, item_table, bias_flat)
